# Initial kernel scaffold; baseline (speedup 1.0000x reference)
import jax
import jax.numpy as jnp
from jax.experimental import pallas as pl


def kernel(x, pairlist, f_ij, rcut_ij, W_in, b_in, W_f, b_f, W_out, b_out):
    raise NotImplementedError("write your pallas kernel here")


# speedup vs baseline: 2.1192x; 2.1192x over previous
import functools

import jax
import jax.numpy as jnp
from jax import lax
from jax.experimental import pallas as pl
from jax.experimental.pallas import tpu as pltpu
from jax.experimental.pallas import tpu_sc as plsc

N_ATOMS = 10000
N_PAIRS = 320000
D = 128
NR_RBF = 20

_LN2 = 0.6931471805599453


def _ssp(t):
    return jnp.maximum(t, 0.0) + jnp.log1p(jnp.exp(-jnp.abs(t))) - _LN2


_HB = 1000


def _h_body(x_ref, w_ref, b_ref, o_ref):
    o_ref[...] = (
        jnp.dot(x_ref[...], w_ref[...], preferred_element_type=jnp.float32)
        + b_ref[...]
    )


def _h_kernel(x, W_in, b_in):
    return pl.pallas_call(
        _h_body,
        grid=(N_ATOMS // _HB,),
        in_specs=[
            pl.BlockSpec((_HB, D), lambda i: (i, 0)),
            pl.BlockSpec((D, D), lambda i: (0, 0)),
            pl.BlockSpec((1, D), lambda i: (0, 0)),
        ],
        out_specs=pl.BlockSpec((_HB, D), lambda i: (i, 0)),
        out_shape=jax.ShapeDtypeStruct((N_ATOMS, D), jnp.float32),
    )(x, W_in, b_in.reshape(1, D))


_EB = 4000


def _wij_body(f_ref, r_ref, w_ref, b_ref, o_ref):
    t = jnp.dot(f_ref[...], w_ref[...], preferred_element_type=jnp.float32)
    t = t + b_ref[...]
    o_ref[...] = _ssp(t) * r_ref[...]


def _wij_kernel(f_ij, rcut_ij, W_f, b_f):
    return pl.pallas_call(
        _wij_body,
        grid=(N_PAIRS // _EB,),
        in_specs=[
            pl.BlockSpec((_EB, NR_RBF), lambda i: (i, 0)),
            pl.BlockSpec((_EB, 1), lambda i: (i, 0)),
            pl.BlockSpec((NR_RBF, D), lambda i: (0, 0)),
            pl.BlockSpec((1, D), lambda i: (0, 0)),
        ],
        out_specs=pl.BlockSpec((_EB, D), lambda i: (i, 0)),
        out_shape=jax.ShapeDtypeStruct((N_PAIRS, D), jnp.float32),
    )(f_ij, rcut_ij.reshape(N_PAIRS, 1), W_f, b_f.reshape(1, D))


_NC = 2
_NS = 16
_NW = _NC * _NS
_EPW = N_PAIRS // _NW
_B = 80
_CHUNKS = _EPW // _B
_NPAD = 10240
_RPS = _NPAD // _NS


def _sc_scatter(h, wij, idx_i, idx_j):
    mesh = plsc.VectorSubcoreMesh(core_axis_name="c", subcore_axis_name="s")

    @functools.partial(
        pl.kernel,
        out_type=jax.ShapeDtypeStruct((_NC, _NPAD, D), jnp.float32),
        mesh=mesh,
        scratch_types=[
            pltpu.VMEM((_B,), jnp.int32),
            pltpu.VMEM((_B,), jnp.int32),
            pltpu.VMEM((_B, D), jnp.float32),
            pltpu.VMEM((_B, D), jnp.float32),
            pltpu.VMEM_SHARED((_NPAD, D), jnp.float32),
        ],
    )
    def k(h_hbm, wij_hbm, ii_hbm, jj_hbm, out_hbm, ii_v, jj_v, rows_v, w_v, acc):
        cid = lax.axis_index("c")
        sid = lax.axis_index("s")

        @pl.loop(0, _B)
        def _(r):
            @pl.loop(0, D, step=16)
            def _(c1):
                rows_v.at[pl.ds(r, 1), pl.ds(c1, 16)][...] = jnp.zeros(
                    (1, 16), jnp.float32
                )

        for t in range(_RPS // _B):
            pltpu.sync_copy(
                rows_v,
                acc.at[pl.ds(sid * _RPS + t * _B, _B)],
            )
        plsc.subcore_barrier()

        base = (cid * _NS + sid) * _EPW

        @pl.loop(0, _CHUNKS)
        def _(kk):
            off = base + kk * _B
            pltpu.sync_copy(ii_hbm.at[pl.ds(off, _B)], ii_v)
            pltpu.sync_copy(jj_hbm.at[pl.ds(off, _B)], jj_v)
            pltpu.sync_copy(h_hbm.at[jj_v], rows_v)
            pltpu.sync_copy(wij_hbm.at[pl.ds(off, _B)], w_v)

            @pl.loop(0, _B)
            def _(r):
                @pl.loop(0, D, step=16)
                def _(c1):
                    s_ = (pl.ds(r, 1), pl.ds(c1, 16))
                    rows_v.at[*s_][...] = rows_v.at[*s_][...] * w_v.at[*s_][...]

            pltpu.sync_copy(rows_v, acc.at[ii_v], add=True)

        plsc.subcore_barrier()
        sl = pl.ds(sid * _RPS, _RPS)
        pltpu.sync_copy(acc.at[sl], out_hbm.at[cid, sl])

    return k(h, wij, idx_i, idx_j)


def _out_body(a_ref, w_ref, b_ref, o_ref):
    s = a_ref[0] + a_ref[1]
    t = jnp.dot(s, w_ref[...], preferred_element_type=jnp.float32) + b_ref[...]
    o_ref[...] = _ssp(t)


def _out_kernel(acc, W_out, b_out):
    return pl.pallas_call(
        _out_body,
        grid=(N_ATOMS // _HB,),
        in_specs=[
            pl.BlockSpec((_NC, _HB, D), lambda i: (0, i, 0)),
            pl.BlockSpec((D, D), lambda i: (0, 0)),
            pl.BlockSpec((1, D), lambda i: (0, 0)),
        ],
        out_specs=pl.BlockSpec((_HB, D), lambda i: (i, 0)),
        out_shape=jax.ShapeDtypeStruct((N_ATOMS, D), jnp.float32),
    )(acc, W_out, b_out.reshape(1, D))


def kernel(x, pairlist, f_ij, rcut_ij, W_in, b_in, W_f, b_f, W_out, b_out):
    idx_i = pairlist[0]
    idx_j = pairlist[1]
    h = _h_kernel(x, W_in, b_in)
    wij = _wij_kernel(f_ij, rcut_ij, W_f, b_f)
    acc = _sc_scatter(h, wij, idx_i, idx_j)
    return _out_kernel(acc, W_out, b_out)


# speedup vs baseline: 2.4760x; 1.1684x over previous
import functools

import jax
import jax.numpy as jnp
from jax import lax
from jax.experimental import pallas as pl
from jax.experimental.pallas import tpu as pltpu
from jax.experimental.pallas import tpu_sc as plsc

N_ATOMS = 10000
N_PAIRS = 320000
D = 128
NR_RBF = 20

_LN2 = 0.6931471805599453


def _ssp(t):
    return jnp.maximum(t, 0.0) + jnp.log1p(jnp.exp(-jnp.abs(t))) - _LN2


_HB = 1000


def _h_body(x_ref, w_ref, b_ref, o_ref):
    o_ref[...] = (
        jnp.dot(x_ref[...], w_ref[...], preferred_element_type=jnp.float32)
        + b_ref[...]
    )


def _h_kernel(x, W_in, b_in):
    return pl.pallas_call(
        _h_body,
        grid=(N_ATOMS // _HB,),
        in_specs=[
            pl.BlockSpec((_HB, D), lambda i: (i, 0)),
            pl.BlockSpec((D, D), lambda i: (0, 0)),
            pl.BlockSpec((1, D), lambda i: (0, 0)),
        ],
        out_specs=pl.BlockSpec((_HB, D), lambda i: (i, 0)),
        out_shape=jax.ShapeDtypeStruct((N_ATOMS, D), jnp.float32),
    )(x, W_in, b_in.reshape(1, D))


_EB = 3200
_RROWS = _EB // D


def _wij_body(f_ref, r_ref, w_ref, b_ref, o_ref):
    t = jnp.dot(f_ref[...], w_ref[...], preferred_element_type=jnp.float32)
    t = t + b_ref[...]
    w = _ssp(t)
    r = r_ref[0]
    ident = (
        lax.broadcasted_iota(jnp.int32, (D, D), 0)
        == lax.broadcasted_iota(jnp.int32, (D, D), 1)
    ).astype(jnp.float32)
    rt = lax.dot_general(
        ident, r, (((1,), (1,)), ((), ())), preferred_element_type=jnp.float32
    )
    for i in range(_RROWS):
        o_ref[i * D : (i + 1) * D, :] = (
            w[i * D : (i + 1) * D, :] * rt[:, i : i + 1]
        )


def _wij_kernel(f_ij, rcut_ij, W_f, b_f):
    return pl.pallas_call(
        _wij_body,
        grid=(N_PAIRS // _EB,),
        in_specs=[
            pl.BlockSpec((_EB, NR_RBF), lambda i: (i, 0)),
            pl.BlockSpec((1, _RROWS, D), lambda i: (i, 0, 0)),
            pl.BlockSpec((NR_RBF, D), lambda i: (0, 0)),
            pl.BlockSpec((1, D), lambda i: (0, 0)),
        ],
        out_specs=pl.BlockSpec((_EB, D), lambda i: (i, 0)),
        out_shape=jax.ShapeDtypeStruct((N_PAIRS, D), jnp.float32),
    )(f_ij, rcut_ij.reshape(N_PAIRS // _EB, _RROWS, D), W_f, b_f.reshape(1, D))


_NC = 2
_NS = 16
_NW = _NC * _NS
_EPW = N_PAIRS // _NW
_B = 80
_CHUNKS = _EPW // _B
_NPAD = 10240
_RPS = _NPAD // _NS


def _sc_scatter(h, wij, idx_i, idx_j):
    mesh = plsc.VectorSubcoreMesh(core_axis_name="c", subcore_axis_name="s")

    @functools.partial(
        pl.kernel,
        out_type=jax.ShapeDtypeStruct((_NC, _NPAD, D), jnp.float32),
        mesh=mesh,
        scratch_types=[
            pltpu.VMEM((_B,), jnp.int32),
            pltpu.VMEM((_B,), jnp.int32),
            pltpu.VMEM((_B, D), jnp.float32),
            pltpu.VMEM((_B, D), jnp.float32),
            pltpu.VMEM_SHARED((_NPAD, D), jnp.float32),
        ],
    )
    def k(h_hbm, wij_hbm, ii_hbm, jj_hbm, out_hbm, ii_v, jj_v, rows_v, w_v, acc):
        cid = lax.axis_index("c")
        sid = lax.axis_index("s")

        @pl.loop(0, _B)
        def _(r):
            @pl.loop(0, D, step=16)
            def _(c1):
                rows_v.at[pl.ds(r, 1), pl.ds(c1, 16)][...] = jnp.zeros(
                    (1, 16), jnp.float32
                )

        for t in range(_RPS // _B):
            pltpu.sync_copy(
                rows_v,
                acc.at[pl.ds(sid * _RPS + t * _B, _B)],
            )
        plsc.subcore_barrier()

        base = (cid * _NS + sid) * _EPW

        @pl.loop(0, _CHUNKS)
        def _(kk):
            off = base + kk * _B
            pltpu.sync_copy(ii_hbm.at[pl.ds(off, _B)], ii_v)
            pltpu.sync_copy(jj_hbm.at[pl.ds(off, _B)], jj_v)
            pltpu.sync_copy(h_hbm.at[jj_v], rows_v)
            pltpu.sync_copy(wij_hbm.at[pl.ds(off, _B)], w_v)

            @pl.loop(0, _B)
            def _(r):
                @pl.loop(0, D, step=16)
                def _(c1):
                    s_ = (pl.ds(r, 1), pl.ds(c1, 16))
                    rows_v.at[*s_][...] = rows_v.at[*s_][...] * w_v.at[*s_][...]

            pltpu.sync_copy(rows_v, acc.at[ii_v], add=True)

        plsc.subcore_barrier()
        sl = pl.ds(sid * _RPS, _RPS)
        pltpu.sync_copy(acc.at[sl], out_hbm.at[cid, sl])

    return k(h, wij, idx_i, idx_j)


def _out_body(a_ref, w_ref, b_ref, o_ref):
    s = a_ref[0] + a_ref[1]
    t = jnp.dot(s, w_ref[...], preferred_element_type=jnp.float32) + b_ref[...]
    o_ref[...] = _ssp(t)


def _out_kernel(acc, W_out, b_out):
    return pl.pallas_call(
        _out_body,
        grid=(N_ATOMS // _HB,),
        in_specs=[
            pl.BlockSpec((_NC, _HB, D), lambda i: (0, i, 0)),
            pl.BlockSpec((D, D), lambda i: (0, 0)),
            pl.BlockSpec((1, D), lambda i: (0, 0)),
        ],
        out_specs=pl.BlockSpec((_HB, D), lambda i: (i, 0)),
        out_shape=jax.ShapeDtypeStruct((N_ATOMS, D), jnp.float32),
    )(acc, W_out, b_out.reshape(1, D))


def kernel(x, pairlist, f_ij, rcut_ij, W_in, b_in, W_f, b_f, W_out, b_out):
    idx_i = pairlist[0]
    idx_j = pairlist[1]
    h = _h_kernel(x, W_in, b_in)
    wij = _wij_kernel(f_ij, rcut_ij, W_f, b_f)
    acc = _sc_scatter(h, wij, idx_i, idx_j)
    return _out_kernel(acc, W_out, b_out)


# speedup vs baseline: 3.5007x; 1.4138x over previous
import functools

import jax
import jax.numpy as jnp
from jax import lax
from jax.experimental import pallas as pl
from jax.experimental.pallas import tpu as pltpu
from jax.experimental.pallas import tpu_sc as plsc

N_ATOMS = 10000
N_PAIRS = 320000
D = 128
NR_RBF = 20

_LN2 = 0.6931471805599453


def _ssp(t):
    return jnp.maximum(t, 0.0) + jnp.log1p(jnp.exp(-jnp.abs(t))) - _LN2


_HB = 1000


def _h_body(x_ref, w_ref, b_ref, o_ref):
    o_ref[...] = (
        jnp.dot(x_ref[...], w_ref[...], preferred_element_type=jnp.float32)
        + b_ref[...]
    )


def _h_kernel(x, W_in, b_in):
    return pl.pallas_call(
        _h_body,
        grid=(N_ATOMS // _HB,),
        in_specs=[
            pl.BlockSpec((_HB, D), lambda i: (i, 0)),
            pl.BlockSpec((D, D), lambda i: (0, 0)),
            pl.BlockSpec((1, D), lambda i: (0, 0)),
        ],
        out_specs=pl.BlockSpec((_HB, D), lambda i: (i, 0)),
        out_shape=jax.ShapeDtypeStruct((N_ATOMS, D), jnp.float32),
    )(x, W_in, b_in.reshape(1, D))


_EB = 3200
_RROWS = _EB // D


def _wij_body(f_ref, r_ref, w_ref, b_ref, o_ref):
    t = jnp.dot(f_ref[...], w_ref[...], preferred_element_type=jnp.float32)
    t = t + b_ref[...]
    w = _ssp(t)
    r = r_ref[0]
    ident = (
        lax.broadcasted_iota(jnp.int32, (D, D), 0)
        == lax.broadcasted_iota(jnp.int32, (D, D), 1)
    ).astype(jnp.float32)
    rt = lax.dot_general(
        ident, r, (((1,), (1,)), ((), ())), preferred_element_type=jnp.float32
    )
    for i in range(_RROWS):
        o_ref[i * D : (i + 1) * D, :] = (
            w[i * D : (i + 1) * D, :] * rt[:, i : i + 1]
        )


def _wij_kernel(f_ij, rcut_ij, W_f, b_f):
    return pl.pallas_call(
        _wij_body,
        grid=(N_PAIRS // _EB,),
        in_specs=[
            pl.BlockSpec((_EB, NR_RBF), lambda i: (i, 0)),
            pl.BlockSpec((1, _RROWS, D), lambda i: (i, 0, 0)),
            pl.BlockSpec((NR_RBF, D), lambda i: (0, 0)),
            pl.BlockSpec((1, D), lambda i: (0, 0)),
        ],
        out_specs=pl.BlockSpec((_EB, D), lambda i: (i, 0)),
        out_shape=jax.ShapeDtypeStruct((N_PAIRS, D), jnp.float32),
    )(f_ij, rcut_ij.reshape(N_PAIRS // _EB, _RROWS, D), W_f, b_f.reshape(1, D))


_NC = 2
_NS = 16
_NW = _NC * _NS
_EPW = N_PAIRS // _NW
_B = 80
_CHUNKS = _EPW // _B
_NPAD = 10240
_RPS = _NPAD // _NS


def _sc_scatter(h, wij, idx_i, idx_j):
    mesh = plsc.VectorSubcoreMesh(core_axis_name="c", subcore_axis_name="s")

    @functools.partial(
        pl.kernel,
        out_type=jax.ShapeDtypeStruct((_NC, _NPAD, D), jnp.float32),
        mesh=mesh,
        scratch_types=[pltpu.VMEM((_B, D), jnp.float32) for _ in range(4)]
        + [pltpu.VMEM((_B,), jnp.int32) for _ in range(8)]
        + [
            pltpu.VMEM_SHARED((_NPAD, D), jnp.float32),
            pltpu.SemaphoreType.DMA((2,)),
            pltpu.SemaphoreType.DMA((2,)),
            pltpu.SemaphoreType.DMA((2,)),
            pltpu.SemaphoreType.DMA((4,)),
        ],
    )
    def k(h_hbm, wij_hbm, ii_hbm, jj_hbm, out_hbm, *rest):
        rows = rest[0:2]
        wv = rest[2:4]
        ii = rest[4:8]
        jj = rest[8:12]
        acc, g_sem, w_sem, s_sem, i_sem = rest[12:]
        cid = lax.axis_index("c")
        sid = lax.axis_index("s")
        wid = cid * _NS + sid
        ebase = wid * _EPW

        @pl.loop(0, _B)
        def _(r):
            @pl.loop(0, D, step=16)
            def _(c1):
                rows[0].at[pl.ds(r, 1), pl.ds(c1, 16)][...] = jnp.zeros(
                    (1, 16), jnp.float32
                )

        for t in range(_RPS // _B):
            pltpu.sync_copy(rows[0], acc.at[pl.ds(sid * _RPS + t * _B, _B)])
        plsc.subcore_barrier()

        def idx_off(c):
            return pl.ds(ebase + c * _B, _B)

        pltpu.sync_copy(ii_hbm.at[idx_off(0)], ii[0])
        pltpu.sync_copy(jj_hbm.at[idx_off(0)], jj[0])

        def mul_scatter(c, q, jm1):
            pltpu.make_async_copy(h_hbm.at[jj[jm1]], rows[q], g_sem.at[q]).wait()
            pltpu.make_async_copy(
                wij_hbm.at[idx_off(c - 1)], wv[q], w_sem.at[q]
            ).wait()

            @pl.loop(0, _B)
            def _(r):
                @pl.loop(0, D, step=16)
                def _(c1):
                    s_ = (pl.ds(r, 1), pl.ds(c1, 16))
                    wv[q].at[*s_][...] = rows[q].at[*s_][...] * wv[q].at[*s_][...]

            pltpu.async_copy(wv[q], acc.at[ii[jm1]], s_sem.at[q], add=True)

        @pl.loop(0, 32)
        def _(t):
            for j in range(4):
                c = t * 4 + j
                p = j % 2
                q = 1 - p
                jm1 = (j + 3) % 4
                jp1 = (j + 1) % 4

                cond_a = (t >= 1) if j == 0 else ((t <= 30) if j >= 2 else None)
                body_a = functools.partial(mul_scatter, c, q, jm1)
                if cond_a is None:
                    body_a()
                else:
                    pl.when(cond_a)(body_a)

                def body_c():
                    pltpu.make_async_copy(
                        ii_hbm.at[idx_off(c)], ii[j], i_sem.at[j]
                    ).wait()
                    pltpu.make_async_copy(
                        jj_hbm.at[idx_off(c)], jj[j], i_sem.at[j]
                    ).wait()
                    pltpu.async_copy(h_hbm.at[jj[j]], rows[p], g_sem.at[p])

                def body_c0():
                    pltpu.async_copy(h_hbm.at[jj[0]], rows[0], g_sem.at[0])

                if j == 0:
                    pl.when(t >= 1)(body_c)
                    pl.when(t == 0)(body_c0)
                else:
                    pl.when(t <= 30)(body_c)

                def body_d():
                    pltpu.make_async_copy(
                        wv[p], acc.at[ii[(j + 2) % 4]], s_sem.at[p]
                    ).wait()

                if j <= 1:
                    pl.when(t >= 1)(body_d)
                else:
                    pl.when(t <= 30)(body_d)

                def body_c2():
                    pltpu.async_copy(
                        wij_hbm.at[idx_off(c)], wv[p], w_sem.at[p]
                    )

                if j == 0:
                    body_c2()
                else:
                    pl.when(t <= 30)(body_c2)

                def body_e():
                    pltpu.async_copy(
                        ii_hbm.at[idx_off(c + 1)], ii[jp1], i_sem.at[jp1]
                    )
                    pltpu.async_copy(
                        jj_hbm.at[idx_off(c + 1)], jj[jp1], i_sem.at[jp1]
                    )

                pl.when(t <= 30)(body_e)

        pltpu.make_async_copy(wv[0], acc.at[ii[0]], s_sem.at[0]).wait()

        plsc.subcore_barrier()
        sl = pl.ds(sid * _RPS, _RPS)
        pltpu.sync_copy(acc.at[sl], out_hbm.at[cid, sl])

    return k(h, wij, idx_i, idx_j)


def _out_body(a_ref, w_ref, b_ref, o_ref):
    s = a_ref[0] + a_ref[1]
    t = jnp.dot(s, w_ref[...], preferred_element_type=jnp.float32) + b_ref[...]
    o_ref[...] = _ssp(t)


def _out_kernel(acc, W_out, b_out):
    return pl.pallas_call(
        _out_body,
        grid=(N_ATOMS // _HB,),
        in_specs=[
            pl.BlockSpec((_NC, _HB, D), lambda i: (0, i, 0)),
            pl.BlockSpec((D, D), lambda i: (0, 0)),
            pl.BlockSpec((1, D), lambda i: (0, 0)),
        ],
        out_specs=pl.BlockSpec((_HB, D), lambda i: (i, 0)),
        out_shape=jax.ShapeDtypeStruct((N_ATOMS, D), jnp.float32),
    )(acc, W_out, b_out.reshape(1, D))


def kernel(x, pairlist, f_ij, rcut_ij, W_in, b_in, W_f, b_f, W_out, b_out):
    idx_i = pairlist[0]
    idx_j = pairlist[1]
    h = _h_kernel(x, W_in, b_in)
    wij = _wij_kernel(f_ij, rcut_ij, W_f, b_f)
    acc = _sc_scatter(h, wij, idx_i, idx_j)
    return _out_kernel(acc, W_out, b_out)
